# baseline (speedup 1.0000x reference)
import functools

import jax
import jax.numpy as jnp
from jax import lax
from jax.experimental import pallas as pl
from jax.experimental.pallas import tpu as pltpu
from jax.experimental.pallas import tpu_sc as plsc

NC = 2
NS = 16
LANES = 16
NW = NC * NS


@functools.partial(jax.jit, static_argnames=("interpret",))
def _w2v(cen_embed, ctx_t, xtab_a, xtab_b, *, interpret=False):
    B, D = cen_embed.shape
    V, H = xtab_a.shape
    L = ctx_t.shape[0]
    assert D == 64 and B % NW == 0
    bpw = B // NW
    C = 32
    assert bpw % (2 * C) == 0
    n_chunks = bpw // C
    rows_per_chunk = C * L

    mesh = plsc.VectorSubcoreMesh(
        core_axis_name="c", subcore_axis_name="s",
        num_cores=NC, num_subcores=NS)

    @functools.partial(
        pl.kernel,
        out_type=jax.ShapeDtypeStruct((L, B), jnp.float32),
        mesh=mesh,
        scratch_types=[
            pltpu.VMEM((L, bpw), jnp.int32),
            pltpu.VMEM((C, D), jnp.float32),
            pltpu.VMEM((C, D), jnp.float32),
            pltpu.VMEM((rows_per_chunk, 32), jnp.float32),
            pltpu.VMEM((rows_per_chunk, 32), jnp.float32),
            pltpu.VMEM((rows_per_chunk, 32), jnp.float32),
            pltpu.VMEM((rows_per_chunk, 32), jnp.float32),
            pltpu.VMEM((L, C), jnp.float32),
            pltpu.VMEM((L, C), jnp.float32),
            pltpu.SemaphoreType.DMA,
            pltpu.SemaphoreType.DMA,
            pltpu.SemaphoreType.DMA,
        ],
        compiler_params=pltpu.CompilerParams(
            needs_layout_passes=False, use_tc_tiling_on_sc=False),
        interpret=interpret,
    )
    def k(cen_hbm, ctx_hbm, xta_hbm, xtb_hbm, out_hbm,
          ctx_idx, cen_a, cen_b, ctx_a, ctx_a2, ctx_b, ctx_b2,
          out_a, out_b, sem_a, sem_b, sem_o):
        wid = lax.axis_index("s") * NC + lax.axis_index("c")
        base = wid * bpw

        pltpu.sync_copy(ctx_hbm.at[:, pl.ds(base, bpw)], ctx_idx)

        def start_gathers(chunk, cen_rows, ctx_lo, ctx_hi, sem):
            off = chunk * C
            pltpu.async_copy(
                cen_hbm.at[pl.ds(base + off, C), :], cen_rows, sem)
            for l in range(L):
                idx = ctx_idx.at[l, pl.ds(off, C)]
                pltpu.async_copy(
                    xta_hbm.at[idx], ctx_lo.at[pl.ds(l * C, C)], sem)
                pltpu.async_copy(
                    xtb_hbm.at[idx], ctx_hi.at[pl.ds(l * C, C)], sem)

        def wait_gathers(cen_rows, ctx_lo, ctx_hi, sem):
            pltpu.make_async_copy(
                cen_hbm.at[pl.ds(0, C), :], cen_rows, sem).wait()
            for l in range(L):
                idx = ctx_idx.at[l, pl.ds(0, C)]
                pltpu.make_async_copy(
                    xta_hbm.at[idx], ctx_lo.at[pl.ds(l * C, C)], sem).wait()
                pltpu.make_async_copy(
                    xtb_hbm.at[idx], ctx_hi.at[pl.ds(l * C, C)], sem).wait()

        def compute(chunk, cen_rows, ctx_lo, ctx_hi, out_v):
            lanes = lax.iota(jnp.int32, LANES)
            for grp in range(C // LANES):
                def grp_body(i16, res):
                    i = grp * LANES + i16
                    mask = lanes == i16
                    cen = [cen_rows[i, pl.ds(16 * kk, 16)]
                           for kk in range(4)]
                    new_res = []
                    for l in range(L):
                        j = l * C + i
                        s = ctx_lo[j, pl.ds(0, 16)] * cen[0]
                        s = s + ctx_lo[j, pl.ds(16, 16)] * cen[1]
                        s = s + ctx_hi[j, pl.ds(0, 16)] * cen[2]
                        s = s + ctx_hi[j, pl.ds(16, 16)] * cen[3]
                        tot = jnp.full((LANES,), jnp.sum(s), jnp.float32)
                        new_res.append(jnp.where(mask, tot, res[l]))
                    return tuple(new_res)

                res = lax.fori_loop(
                    0, LANES, grp_body,
                    tuple(jnp.zeros((LANES,), jnp.float32)
                          for _ in range(L)))
                for l in range(L):
                    out_v[l, pl.ds(grp * LANES, LANES)] = res[l]
            pltpu.async_copy(
                out_v, out_hbm.at[:, pl.ds(base + chunk * C, C)], sem_o)

        def wait_out(out_v, chunk):
            pltpu.make_async_copy(
                out_v, out_hbm.at[:, pl.ds(base + chunk * C, C)],
                sem_o).wait()

        start_gathers(0, cen_a, ctx_a, ctx_a2, sem_a)

        def pair_body(g, _):
            c0 = 2 * g
            start_gathers(c0 + 1, cen_b, ctx_b, ctx_b2, sem_b)
            wait_gathers(cen_a, ctx_a, ctx_a2, sem_a)

            @pl.when(g > 0)
            def _w():
                wait_out(out_a, c0 - 2)
            compute(c0, cen_a, ctx_a, ctx_a2, out_a)

            @pl.when(c0 + 2 < n_chunks)
            def _s():
                start_gathers(c0 + 2, cen_a, ctx_a, ctx_a2, sem_a)
            wait_gathers(cen_b, ctx_b, ctx_b2, sem_b)

            @pl.when(g > 0)
            def _w2():
                wait_out(out_b, c0 - 1)
            compute(c0 + 1, cen_b, ctx_b, ctx_b2, out_b)
            return _

        lax.fori_loop(0, n_chunks // 2, pair_body, None)
        wait_out(out_a, n_chunks - 2)
        wait_out(out_b, n_chunks - 1)

    return k(cen_embed, ctx_t, xtab_a, xtab_b)


def kernel(center, context, center_table, context_table):
    B = center.shape[0]
    L = context.shape[1]
    cen_embed = jnp.take(center_table, center, axis=0)
    out_t = _w2v(cen_embed, context.T,
                 context_table[:, :32], context_table[:, 32:])
    return out_t.T


# speedup vs baseline: 1.8887x; 1.8887x over previous
import functools

import jax
import jax.numpy as jnp
from jax import lax
from jax.experimental import pallas as pl
from jax.experimental.pallas import tpu as pltpu
from jax.experimental.pallas import tpu_sc as plsc

NC = 2
NS = 16
LANES = 16
NW = NC * NS


@functools.partial(jax.jit, static_argnames=("interpret",))
def _w2v(cen_embed, ctx_t, context_table, *, interpret=False):
    B, D = cen_embed.shape
    V, _ = context_table.shape
    L = ctx_t.shape[0]
    assert D == 64 and B % NW == 0
    bpw = B // NW
    C = 32
    assert bpw % (2 * C) == 0
    n_chunks = bpw // C
    rows_per_chunk = C * L

    mesh = plsc.VectorSubcoreMesh(
        core_axis_name="c", subcore_axis_name="s",
        num_cores=NC, num_subcores=NS)

    @functools.partial(
        pl.kernel,
        out_type=jax.ShapeDtypeStruct((L, B), jnp.float32),
        mesh=mesh,
        scratch_types=[
            pltpu.VMEM((L, bpw), jnp.int32),
            pltpu.VMEM((C, D), jnp.float32),
            pltpu.VMEM((C, D), jnp.float32),
            pltpu.VMEM((rows_per_chunk, D), jnp.float32),
            pltpu.VMEM((rows_per_chunk, D), jnp.float32),
            pltpu.VMEM((L, C), jnp.float32),
            pltpu.VMEM((L, C), jnp.float32),
            pltpu.SemaphoreType.DMA,
            pltpu.SemaphoreType.DMA,
            pltpu.SemaphoreType.DMA,
        ],
        compiler_params=pltpu.CompilerParams(
            needs_layout_passes=False, use_tc_tiling_on_sc=False),
        interpret=interpret,
    )
    def k(cen_hbm, ctx_hbm, xtab_hbm, out_hbm,
          ctx_idx, cen_a, cen_b, ctx_a, ctx_b, out_a, out_b,
          sem_a, sem_b, sem_o):
        wid = lax.axis_index("s") * NC + lax.axis_index("c")
        base = wid * bpw

        pltpu.sync_copy(ctx_hbm.at[:, pl.ds(base, bpw)], ctx_idx)

        def start_gathers(chunk, cen_rows, ctx_rows, sem):
            off = chunk * C
            pltpu.async_copy(
                cen_hbm.at[pl.ds(base + off, C), :], cen_rows, sem)
            for l in range(L):
                pltpu.async_copy(
                    xtab_hbm.at[ctx_idx.at[l, pl.ds(off, C)]],
                    ctx_rows.at[pl.ds(l * C, C)], sem)

        def wait_gathers(cen_rows, ctx_rows, sem):
            pltpu.make_async_copy(
                cen_hbm.at[pl.ds(0, C), :], cen_rows, sem).wait()
            for l in range(L):
                pltpu.make_async_copy(
                    xtab_hbm.at[ctx_idx.at[l, pl.ds(0, C)]],
                    ctx_rows.at[pl.ds(l * C, C)], sem).wait()

        def compute(chunk, cen_rows, ctx_rows, out_v):
            lanes = lax.iota(jnp.int32, LANES)
            for grp in range(C // LANES):
                def grp_body(i16, res):
                    i = grp * LANES + i16
                    mask = lanes == i16
                    cen = [cen_rows[i, pl.ds(16 * kk, 16)]
                           for kk in range(4)]
                    new_res = []
                    for l in range(L):
                        j = l * C + i
                        s = ctx_rows[j, pl.ds(0, 16)] * cen[0]
                        for kk in range(1, 4):
                            s = s + ctx_rows[j, pl.ds(16 * kk, 16)] * cen[kk]
                        tot = jnp.full((LANES,), jnp.sum(s), jnp.float32)
                        new_res.append(jnp.where(mask, tot, res[l]))
                    return tuple(new_res)

                res = lax.fori_loop(
                    0, LANES, grp_body,
                    tuple(jnp.zeros((LANES,), jnp.float32)
                          for _ in range(L)))
                for l in range(L):
                    out_v[l, pl.ds(grp * LANES, LANES)] = res[l]
            pltpu.async_copy(
                out_v, out_hbm.at[:, pl.ds(base + chunk * C, C)], sem_o)

        def wait_out(out_v, chunk):
            pltpu.make_async_copy(
                out_v, out_hbm.at[:, pl.ds(base + chunk * C, C)],
                sem_o).wait()

        start_gathers(0, cen_a, ctx_a, sem_a)

        def pair_body(g, _):
            c0 = 2 * g
            start_gathers(c0 + 1, cen_b, ctx_b, sem_b)
            wait_gathers(cen_a, ctx_a, sem_a)

            @pl.when(g > 0)
            def _w():
                wait_out(out_a, c0 - 2)
            compute(c0, cen_a, ctx_a, out_a)

            @pl.when(c0 + 2 < n_chunks)
            def _s():
                start_gathers(c0 + 2, cen_a, ctx_a, sem_a)
            wait_gathers(cen_b, ctx_b, sem_b)

            @pl.when(g > 0)
            def _w2():
                wait_out(out_b, c0 - 1)
            compute(c0 + 1, cen_b, ctx_b, out_b)
            return _

        lax.fori_loop(0, n_chunks // 2, pair_body, None)
        wait_out(out_a, n_chunks - 2)
        wait_out(out_b, n_chunks - 1)

    return k(cen_embed, ctx_t, context_table)


def kernel(center, context, center_table, context_table):
    B = center.shape[0]
    L = context.shape[1]
    cen_embed = jnp.take(center_table, center, axis=0)
    out_t = _w2v(cen_embed, context.T, context_table)
    return out_t.T
